# baseline (speedup 1.0000x reference)
import functools

import jax
import jax.numpy as jnp
from jax.experimental import pallas as pl
import jax.experimental.pallas.tpu as pltpu

N_EMBED = 1024
E = 8
EP = 128
TOP_K = 2
B_ = 4
S_ = 8192
BS = 512
NS = S_ // BS


def _router_kernel(x_ref, avg_ref, wr_ref, br_ref, wn_ref, bn_ref, ns_ref,
                   router_ref, idx_ref, acc_ref):
    b = pl.program_id(0)
    s = pl.program_id(1)

    @pl.when((b == 0) & (s == 0))
    def _init():
        acc_ref[...] = jnp.zeros_like(acc_ref)

    x = x_ref[0]
    y = jnp.dot(x, wn_ref[...], preferred_element_type=jnp.float32)
    y = y + bn_ref[...]
    sp = jnp.maximum(y, 0.0) + jnp.log1p(jnp.exp(-jnp.abs(y)))
    part = jnp.sum(sp, axis=0, keepdims=True)
    acc_ref[pl.ds(b, 1), :] += part

    @pl.when((b == B_ - 1) & (s == NS - 1))
    def _epilogue():
        mean = acc_ref[...] * (1.0 / S_)
        logits = jnp.dot(avg_ref[...], wr_ref[...],
                         preferred_element_type=jnp.float32) + br_ref[...]
        noisy = logits + ns_ref[...] * mean
        col = jax.lax.broadcasted_iota(jnp.int32, (B_, EP), 1)
        valid = col < E
        neg = jnp.float32(-1e30)
        noisy = jnp.where(valid, noisy, neg)
        m1 = jnp.max(noisy, axis=1, keepdims=True)
        i1 = jnp.min(jnp.where(noisy == m1, col, EP), axis=1, keepdims=True)
        rest = jnp.where(col == i1, neg, noisy)
        m2 = jnp.max(rest, axis=1, keepdims=True)
        i2 = jnp.min(jnp.where(rest == m2, col, EP), axis=1, keepdims=True)
        d = jnp.exp(m2 - m1)
        p1 = 1.0 / (1.0 + d)
        p2 = d / (1.0 + d)
        router = jnp.where(col == i1, p1, jnp.where(col == i2, p2, 0.0))
        router_ref[...] = router
        idx_ref[...] = jnp.where(col == 0, i1, jnp.where(col == 1, i2, 0))


def kernel(mh_output, mh_output_avg, W_route, b_route, W_noise, b_noise):
    noise_sample = jax.random.normal(jax.random.key(42), (B_, E),
                                     dtype=jnp.float32)
    zpad = jnp.zeros((N_EMBED, EP), jnp.float32)
    wn = zpad.at[:, :E].set(W_noise.T)
    wr = zpad.at[:, :E].set(W_route.T)
    bn = jnp.zeros((1, EP), jnp.float32).at[0, :E].set(b_noise)
    br = jnp.zeros((1, EP), jnp.float32).at[0, :E].set(b_route)
    ns = jnp.zeros((B_, EP), jnp.float32).at[:, :E].set(noise_sample)

    router_p, idx_p = pl.pallas_call(
        _router_kernel,
        grid=(B_, NS),
        in_specs=[
            pl.BlockSpec((1, BS, N_EMBED), lambda b, s: (b, s, 0)),
            pl.BlockSpec((B_, N_EMBED), lambda b, s: (0, 0)),
            pl.BlockSpec((N_EMBED, EP), lambda b, s: (0, 0)),
            pl.BlockSpec((1, EP), lambda b, s: (0, 0)),
            pl.BlockSpec((N_EMBED, EP), lambda b, s: (0, 0)),
            pl.BlockSpec((1, EP), lambda b, s: (0, 0)),
            pl.BlockSpec((B_, EP), lambda b, s: (0, 0)),
        ],
        out_specs=[
            pl.BlockSpec((B_, EP), lambda b, s: (0, 0)),
            pl.BlockSpec((B_, EP), lambda b, s: (0, 0)),
        ],
        out_shape=[
            jax.ShapeDtypeStruct((B_, EP), jnp.float32),
            jax.ShapeDtypeStruct((B_, EP), jnp.int32),
        ],
        scratch_shapes=[pltpu.VMEM((B_, EP), jnp.float32)],
    )(mh_output, mh_output_avg, wr, br, wn, bn, ns)

    return router_p[:, :E], idx_p[:, :TOP_K]


# speedup vs baseline: 1.0027x; 1.0027x over previous
import jax
import jax.numpy as jnp
from jax.experimental import pallas as pl
import jax.experimental.pallas.tpu as pltpu

N_EMBED = 1024
E = 8
EP = 128
TOP_K = 2
B_ = 4
S_ = 8192
BS = 512
NS = S_ // BS


def _router_kernel(x_ref, avg_ref, wr_ref, br_ref, wn_ref, bn_ref, ns_ref,
                   router_ref, idx_ref, acc_ref):
    b = pl.program_id(0)
    s = pl.program_id(1)

    @pl.when((b == 0) & (s == 0))
    def _init():
        acc_ref[...] = jnp.zeros_like(acc_ref)

    x = x_ref[0]
    y = jnp.dot(x, wn_ref[...], preferred_element_type=jnp.float32)
    yt = jnp.transpose(y)[:E, :]
    yt = yt + bn_ref[...]
    sp = jnp.maximum(yt, 0.0) + jnp.log1p(jnp.exp(-jnp.abs(yt)))
    part = jnp.sum(sp, axis=1, keepdims=True)
    lane = jax.lax.broadcasted_iota(jnp.int32, (E, EP), 1)
    acc_ref[...] += jnp.where(lane == b, part, 0.0)

    @pl.when((b == B_ - 1) & (s == NS - 1))
    def _epilogue():
        mean = jnp.transpose(acc_ref[...])[:B_, :] * (1.0 / S_)
        logits = (jnp.dot(avg_ref[...], wr_ref[...],
                          preferred_element_type=jnp.float32))[:, :E]
        noisy = logits + br_ref[...] + ns_ref[...] * mean
        col = jax.lax.broadcasted_iota(jnp.int32, (B_, E), 1)
        neg = jnp.float32(-1e30)
        m1 = jnp.max(noisy, axis=1, keepdims=True)
        i1 = jnp.min(jnp.where(noisy == m1, col, E), axis=1, keepdims=True)
        rest = jnp.where(col == i1, neg, noisy)
        m2 = jnp.max(rest, axis=1, keepdims=True)
        i2 = jnp.min(jnp.where(rest == m2, col, E), axis=1, keepdims=True)
        d = jnp.exp(m2 - m1)
        p1 = 1.0 / (1.0 + d)
        p2 = d / (1.0 + d)
        router_ref[:, :E] = jnp.where(col == i1, p1,
                                      jnp.where(col == i2, p2, 0.0))
        idx_ref[:, 0:1] = i1
        idx_ref[:, 1:2] = i2


def kernel(mh_output, mh_output_avg, W_route, b_route, W_noise, b_noise):
    noise_sample = jax.random.normal(jax.random.key(42), (B_, E),
                                     dtype=jnp.float32)
    zpad = jnp.zeros((N_EMBED, EP), jnp.float32)
    wn = zpad.at[:, :E].set(W_noise.T)
    wr = zpad.at[:, :E].set(W_route.T)
    bn = b_noise[:, None]
    br = b_route[None, :]
    ns = noise_sample

    router_p, idx_p = pl.pallas_call(
        _router_kernel,
        grid=(B_, NS),
        in_specs=[
            pl.BlockSpec((1, BS, N_EMBED), lambda b, s: (b, s, 0)),
            pl.BlockSpec((B_, N_EMBED), lambda b, s: (0, 0)),
            pl.BlockSpec((N_EMBED, EP), lambda b, s: (0, 0)),
            pl.BlockSpec((1, E), lambda b, s: (0, 0)),
            pl.BlockSpec((N_EMBED, EP), lambda b, s: (0, 0)),
            pl.BlockSpec((E, 1), lambda b, s: (0, 0)),
            pl.BlockSpec((B_, E), lambda b, s: (0, 0)),
        ],
        out_specs=[
            pl.BlockSpec((B_, EP), lambda b, s: (0, 0)),
            pl.BlockSpec((B_, EP), lambda b, s: (0, 0)),
        ],
        out_shape=[
            jax.ShapeDtypeStruct((B_, EP), jnp.float32),
            jax.ShapeDtypeStruct((B_, EP), jnp.int32),
        ],
        scratch_shapes=[pltpu.VMEM((E, EP), jnp.float32)],
    )(mh_output, mh_output_avg, wr, br, wn, bn, ns)

    return router_p[:, :E], idx_p[:, :TOP_K]


# speedup vs baseline: 1.7357x; 1.7311x over previous
import jax
import jax.numpy as jnp
from jax.experimental import pallas as pl
import jax.experimental.pallas.tpu as pltpu

N_EMBED = 1024
E = 8
EP = 128
TOP_K = 2
B_ = 4
S_ = 8192
BS = 4096
NS = S_ // BS

def _router_kernel(x_ref, avg_ref, wr_ref, br_ref, wn_ref, bn_ref, ns_ref,
                   router_ref, idx_ref, acc_ref):
    b = pl.program_id(0)
    s = pl.program_id(1)

    @pl.when((b == 0) & (s == 0))
    def _init():
        acc_ref[...] = jnp.zeros_like(acc_ref)

    x = x_ref[0]
    yt = jax.lax.dot_general(wn_ref[...], x,
                             (((1,), (1,)), ((), ())),
                             preferred_element_type=jnp.float32)
    yt = yt + bn_ref[...]
    sp = jnp.maximum(yt, 0.0) + jnp.log1p(jnp.exp(-jnp.abs(yt)))
    part = jnp.sum(sp, axis=1, keepdims=True)
    lane = jax.lax.broadcasted_iota(jnp.int32, (E, EP), 1)
    acc_ref[...] += jnp.where(lane == b, part, 0.0)

    @pl.when((b == B_ - 1) & (s == NS - 1))
    def _epilogue():
        mean = jnp.transpose(acc_ref[...])[:B_, :E] * (1.0 / S_)
        logits = jax.lax.dot_general(avg_ref[...], wr_ref[...],
                                     (((1,), (1,)), ((), ())),
                                     preferred_element_type=jnp.float32)
        noisy = logits + br_ref[...] + ns_ref[...] * mean
        col = jax.lax.broadcasted_iota(jnp.int32, (B_, E), 1)
        neg = jnp.float32(-1e30)
        m1 = jnp.max(noisy, axis=1, keepdims=True)
        i1 = jnp.min(jnp.where(noisy == m1, col, E), axis=1, keepdims=True)
        rest = jnp.where(col == i1, neg, noisy)
        m2 = jnp.max(rest, axis=1, keepdims=True)
        i2 = jnp.min(jnp.where(rest == m2, col, E), axis=1, keepdims=True)
        d = jnp.exp(m2 - m1)
        p1 = 1.0 / (1.0 + d)
        p2 = d / (1.0 + d)
        router_ref[...] = jnp.where(col == i1, p1,
                                    jnp.where(col == i2, p2, 0.0))
        idx_ref[:, 0:1] = i1
        idx_ref[:, 1:2] = i2


def kernel(mh_output, mh_output_avg, W_route, b_route, W_noise, b_noise):
    noise_sample = jax.random.normal(jax.random.key(42), (B_, E),
                                     dtype=jnp.float32)
    return pl.pallas_call(
        _router_kernel,
        grid=(B_, NS),
        in_specs=[
            pl.BlockSpec((1, BS, N_EMBED), lambda b, s: (b, s, 0)),
            pl.BlockSpec((B_, N_EMBED), lambda b, s: (0, 0)),
            pl.BlockSpec((E, N_EMBED), lambda b, s: (0, 0)),
            pl.BlockSpec((1, E), lambda b, s: (0, 0)),
            pl.BlockSpec((E, N_EMBED), lambda b, s: (0, 0)),
            pl.BlockSpec((E, 1), lambda b, s: (0, 0)),
            pl.BlockSpec((B_, E), lambda b, s: (0, 0)),
        ],
        out_specs=[
            pl.BlockSpec((B_, E), lambda b, s: (0, 0)),
            pl.BlockSpec((B_, TOP_K), lambda b, s: (0, 0)),
        ],
        out_shape=[
            jax.ShapeDtypeStruct((B_, E), jnp.float32),
            jax.ShapeDtypeStruct((B_, TOP_K), jnp.int32),
        ],
        scratch_shapes=[pltpu.VMEM((E, EP), jnp.float32)],
    )(mh_output, mh_output_avg, W_route, b_route[None, :], W_noise,
      b_noise[:, None], noise_sample)


# speedup vs baseline: 1.7888x; 1.0306x over previous
import jax
import jax.numpy as jnp
from jax.experimental import pallas as pl
import jax.experimental.pallas.tpu as pltpu

N_EMBED = 1024
E = 8
EP = 128
TOP_K = 2
B_ = 4
S_ = 8192
BS = 2048
NS = S_ // BS

def _router_kernel(x_ref, avg_ref, wr_ref, br_ref, wn_ref, bn_ref, ns_ref,
                   router_ref, idx_ref, acc_ref):
    b = pl.program_id(0)
    s = pl.program_id(1)

    @pl.when((b == 0) & (s == 0))
    def _init():
        acc_ref[...] = jnp.zeros_like(acc_ref)

    x = x_ref[0]
    yt = jax.lax.dot_general(wn_ref[...], x,
                             (((1,), (1,)), ((), ())),
                             preferred_element_type=jnp.float32)
    yt = yt + bn_ref[...]
    sp = jnp.maximum(yt, 0.0) + jnp.log1p(jnp.exp(-jnp.abs(yt)))
    part = jnp.sum(sp, axis=1, keepdims=True)
    lane = jax.lax.broadcasted_iota(jnp.int32, (E, EP), 1)
    acc_ref[...] += jnp.where(lane == b, part, 0.0)

    @pl.when((b == B_ - 1) & (s == NS - 1))
    def _epilogue():
        mean = jnp.transpose(acc_ref[...])[:B_, :E] * (1.0 / S_)
        logits = jax.lax.dot_general(avg_ref[...], wr_ref[...],
                                     (((1,), (1,)), ((), ())),
                                     preferred_element_type=jnp.float32)
        noisy = logits + br_ref[...] + ns_ref[...] * mean
        col = jax.lax.broadcasted_iota(jnp.int32, (B_, E), 1)
        neg = jnp.float32(-1e30)
        m1 = jnp.max(noisy, axis=1, keepdims=True)
        i1 = jnp.min(jnp.where(noisy == m1, col, E), axis=1, keepdims=True)
        rest = jnp.where(col == i1, neg, noisy)
        m2 = jnp.max(rest, axis=1, keepdims=True)
        i2 = jnp.min(jnp.where(rest == m2, col, E), axis=1, keepdims=True)
        d = jnp.exp(m2 - m1)
        p1 = 1.0 / (1.0 + d)
        p2 = d / (1.0 + d)
        router_ref[...] = jnp.where(col == i1, p1,
                                    jnp.where(col == i2, p2, 0.0))
        idx_ref[:, 0:1] = i1
        idx_ref[:, 1:2] = i2


def kernel(mh_output, mh_output_avg, W_route, b_route, W_noise, b_noise):
    noise_sample = jax.random.normal(jax.random.key(42), (B_, E),
                                     dtype=jnp.float32)
    return pl.pallas_call(
        _router_kernel,
        grid=(B_, NS),
        in_specs=[
            pl.BlockSpec((1, BS, N_EMBED), lambda b, s: (b, s, 0)),
            pl.BlockSpec((B_, N_EMBED), lambda b, s: (0, 0)),
            pl.BlockSpec((E, N_EMBED), lambda b, s: (0, 0)),
            pl.BlockSpec((1, E), lambda b, s: (0, 0)),
            pl.BlockSpec((E, N_EMBED), lambda b, s: (0, 0)),
            pl.BlockSpec((E, 1), lambda b, s: (0, 0)),
            pl.BlockSpec((B_, E), lambda b, s: (0, 0)),
        ],
        out_specs=[
            pl.BlockSpec((B_, E), lambda b, s: (0, 0)),
            pl.BlockSpec((B_, TOP_K), lambda b, s: (0, 0)),
        ],
        out_shape=[
            jax.ShapeDtypeStruct((B_, E), jnp.float32),
            jax.ShapeDtypeStruct((B_, TOP_K), jnp.int32),
        ],
        scratch_shapes=[pltpu.VMEM((E, EP), jnp.float32)],
    )(mh_output, mh_output_avg, W_route, b_route[None, :], W_noise,
      b_noise[:, None], noise_sample)
